# Initial kernel scaffold; baseline (speedup 1.0000x reference)
import jax
import jax.numpy as jnp
from jax.experimental import pallas as pl


def kernel(params, edge_weight, u, si, ti, edge_index):
    raise NotImplementedError("write your pallas kernel here")


# speedup vs baseline: 9.2309x; 9.2309x over previous
import functools

import jax
import jax.numpy as jnp
from jax import lax
from jax.experimental import pallas as pl
from jax.experimental.pallas import tpu as pltpu
from jax.experimental.pallas import tpu_sc as plsc

_NC = 2
_NS = 16
_NW = _NC * _NS
_STREAM = 128
_KCH = 8
_CH = _KCH * _STREAM


def _sc_mesh():
    return plsc.VectorSubcoreMesh(core_axis_name="c", subcore_axis_name="s",
                                  num_cores=_NC, num_subcores=_NS)


def _build_msgpass(n_pad, groups):
    rpt = n_pad // _NS

    @functools.partial(
        pl.kernel,
        out_type=jax.ShapeDtypeStruct((2, n_pad, 16), jnp.float32),
        mesh=_sc_mesh(),
        compiler_params=pltpu.CompilerParams(use_tc_tiling_on_sc=False),
        scratch_types=[
            pltpu.VMEM_SHARED((n_pad, 16), jnp.float32),
            pltpu.VMEM((_KCH, _STREAM), jnp.int32),
            pltpu.VMEM((_KCH, _STREAM), jnp.int32),
            pltpu.VMEM((_KCH, _STREAM), jnp.float32),
            pltpu.VMEM((_KCH, _STREAM, 16), jnp.float32),
            pltpu.SemaphoreType.DMA,
        ],
        name="sc_msgpass",
    )
    def msgpass(h_a, h_b, zeros_ref, src2d, dst2d, w2d, out,
                acc, sidx, didx, wv, rows, sem):
        c = lax.axis_index("c")
        s = lax.axis_index("s")

        pltpu.sync_copy(zeros_ref.at[pl.ds(s * rpt, rpt)],
                        acc.at[pl.ds(s * rpt, rpt)])
        plsc.subcore_barrier()

        row0 = s * groups * _KCH

        @pl.loop(0, groups)
        def _group(g):
            r0 = row0 + g * _KCH
            pltpu.sync_copy(src2d.at[pl.ds(r0, _KCH)], sidx)
            pltpu.sync_copy(dst2d.at[pl.ds(r0, _KCH)], didx)
            pltpu.sync_copy(w2d.at[pl.ds(r0, _KCH)], wv)
            for j in range(_KCH):
                @pl.when(c == 0)
                def _fire_a(j=j):
                    pltpu.async_copy(h_a.at[sidx.at[j]], rows.at[j], sem)

                @pl.when(c == 1)
                def _fire_b(j=j):
                    pltpu.async_copy(h_b.at[sidx.at[j]], rows.at[j], sem)
            for j in range(_KCH):
                pltpu.make_async_copy(h_a.at[sidx.at[j]], rows.at[j], sem).wait()
            for j in range(_KCH):
                @pl.loop(0, _STREAM // 16)
                def _mul(i16, j=j):
                    base = i16 * 16
                    w16 = wv[j, pl.ds(base, 16)]
                    for l in range(16):
                        wb = jnp.take_along_axis(
                            w16, jnp.full((16,), l, jnp.int32), axis=0)
                        rows[j, base + l, :] = rows[j, base + l, :] * wb
            for j in range(_KCH):
                pltpu.sync_copy(rows.at[j], acc.at[didx.at[j]], add=True)

        plsc.subcore_barrier()
        pltpu.sync_copy(acc.at[pl.ds(s * rpt, rpt)],
                        out.at[c].at[pl.ds(s * rpt, rpt)])

    return msgpass


def _build_gather_cat(bsz):
    per_tile = bsz // _NW

    @functools.partial(
        pl.kernel,
        out_type=jax.ShapeDtypeStruct((3, bsz, 128), jnp.float32),
        mesh=_sc_mesh(),
        compiler_params=pltpu.CompilerParams(use_tc_tiling_on_sc=False),
        scratch_types=[
            pltpu.VMEM((per_tile,), jnp.int32),
            pltpu.VMEM((8, per_tile, 16), jnp.float32),
            pltpu.VMEM((per_tile, 128), jnp.float32),
            pltpu.SemaphoreType.DMA,
        ],
        name="sc_gather_cat",
    )
    def gather_cat(p0, p1, p2, p3, p4, p5, p6, p7, gidx_flat, out,
                   idxv, tmp, buf, sem):
        c = lax.axis_index("c")
        s = lax.axis_index("s")
        wid = s * _NC + c
        parts = (p0, p1, p2, p3, p4, p5, p6, p7)

        @pl.loop(0, 3)
        def _set(jset):
            pltpu.sync_copy(
                gidx_flat.at[pl.ds((jset * _NW + wid) * per_tile, per_tile)],
                idxv)
            for p in range(8):
                pltpu.async_copy(parts[p].at[idxv], tmp.at[p], sem)
            for p in range(8):
                pltpu.make_async_copy(parts[p].at[idxv], tmp.at[p], sem).wait()
            for p in range(8):
                @pl.loop(0, per_tile)
                def _cp(i, p=p):
                    buf[i, pl.ds(p * 16, 16)] = tmp[p, i, :]
            pltpu.sync_copy(buf, out.at[jset].at[pl.ds(wid * per_tile, per_tile)])

    return gather_cat


def _tc_layer(agg_a, agg_b, w, b):
    n = agg_a.shape[0]
    r = 4000
    grid = n // r

    def body(a_ref, b_ref, w_ref, bias_ref, oa_ref, ob_ref):
        x = jnp.concatenate([a_ref[...], b_ref[...]], axis=1)
        h = jnp.dot(x, w_ref[...], preferred_element_type=jnp.float32)
        h = jnp.maximum(h + bias_ref[...], 0.0)
        oa_ref[...] = h[:, :16]
        ob_ref[...] = h[:, 16:]

    return pl.pallas_call(
        body,
        grid=(grid,),
        in_specs=[
            pl.BlockSpec((r, 16), lambda i: (i, 0)),
            pl.BlockSpec((r, 16), lambda i: (i, 0)),
            pl.BlockSpec((32, 32), lambda i: (0, 0)),
            pl.BlockSpec((1, 32), lambda i: (0, 0)),
        ],
        out_specs=[
            pl.BlockSpec((r, 16), lambda i: (i, 0)),
            pl.BlockSpec((r, 16), lambda i: (i, 0)),
        ],
        out_shape=[jax.ShapeDtypeStruct((n, 16), jnp.float32)] * 2,
        name="tc_gnn_layer",
    )(agg_a, agg_b, w, b)


def _tc_mlp(ug, sig, tig, ws, bs, wt, bt, dws, dbs, dwt, dbt):
    bsz = ug.shape[0]
    rb = 512
    grid = bsz // rb

    def body(u_ref, s_ref, t_ref,
             ws0, ws1, ws2, bs0, bs1, bs2,
             wt0, wt1, wt2, bt0, bt1, bt2,
             dws_ref, dbs_ref, dwt_ref, dbt_ref, out_ref):
        u = u_ref[...]
        xs = jnp.concatenate([u, s_ref[...]], axis=1)
        xt = jnp.concatenate([u, t_ref[...]], axis=1)
        for wr, br in ((ws0, bs0), (ws1, bs1), (ws2, bs2)):
            xs = jnp.maximum(
                jnp.dot(xs, wr[...], preferred_element_type=jnp.float32) + br[...], 0.0)
        for wr, br in ((wt0, bt0), (wt1, bt1), (wt2, bt2)):
            xt = jnp.maximum(
                jnp.dot(xt, wr[...], preferred_element_type=jnp.float32) + br[...], 0.0)
        ss = jnp.dot(xs, dws_ref[...], preferred_element_type=jnp.float32) + dbs_ref[...]
        st = jnp.dot(xt, dwt_ref[...], preferred_element_type=jnp.float32) + dbt_ref[...]
        out_ref[...] = jnp.concatenate([ss, st], axis=1)

    def wspec(shape):
        return pl.BlockSpec(shape, lambda i: tuple(0 for _ in shape))

    in_specs = [pl.BlockSpec((rb, 128), lambda i: (i, 0))] * 3
    in_specs += [wspec(w.shape) for w in ws]
    in_specs += [wspec(b.shape) for b in bs]
    in_specs += [wspec(w.shape) for w in wt]
    in_specs += [wspec(b.shape) for b in bt]
    in_specs += [wspec(dws.shape), wspec(dbs.shape), wspec(dwt.shape), wspec(dbt.shape)]

    return pl.pallas_call(
        body,
        grid=(grid,),
        in_specs=in_specs,
        out_specs=pl.BlockSpec((rb, 2), lambda i: (i, 0)),
        out_shape=jax.ShapeDtypeStruct((bsz, 2), jnp.float32),
        name="tc_ncf_mlp",
    )(ug, sig, tig, *ws, *bs, *wt, *bt, dws, dbs, dwt, dbt)


def kernel(params, edge_weight, u, si, ti, edge_index):
    user_n = params["user_emb"].shape[0]
    i1_n = params["item_s_emb"].shape[0]
    n_nodes = user_n + i1_n + params["item_t_emb"].shape[0]
    e = edge_index.shape[1]
    bsz = u.shape[0]

    per_sub = -(-e // (_NS * _CH)) * _CH
    e_pad = per_sub * _NS
    groups = per_sub // _CH
    pad = e_pad - e

    src = edge_index[0]
    dst = edge_index[1]
    src2d = jnp.concatenate([src, jnp.zeros((pad,), src.dtype)]).reshape(-1, _STREAM)
    dst2d = jnp.concatenate([dst, jnp.zeros((pad,), dst.dtype)]).reshape(-1, _STREAM)
    w2d = jnp.concatenate(
        [edge_weight, jnp.zeros((pad,), edge_weight.dtype)]).reshape(-1, _STREAM)

    ego = jnp.concatenate(
        [params["user_emb"], params["item_s_emb"], params["item_t_emb"]], axis=0)
    h_a, h_b = ego[:, :16], ego[:, 16:]

    n_pad = -(-n_nodes // (_NS * 8)) * (_NS * 8)
    zeros_n16 = jnp.zeros((n_pad, 16), jnp.float32)
    msgpass = _build_msgpass(n_pad, groups)

    parts = [h_a, h_b]
    for k in range(len(params["gnn_W"])):
        agg = msgpass(h_a, h_b, zeros_n16, src2d, dst2d, w2d)
        h_a, h_b = _tc_layer(agg[0, :n_nodes], agg[1, :n_nodes],
                             params["gnn_W"][k],
                             params["gnn_b"][k].reshape(1, -1))
        parts += [h_a, h_b]

    gidx_flat = jnp.stack([u, si + user_n, ti + user_n + i1_n]).reshape(-1)
    gath = _build_gather_cat(bsz)(*parts, gidx_flat)

    return _tc_mlp(
        gath[0], gath[1], gath[2],
        params["ncf_s_W"], [b.reshape(1, -1) for b in params["ncf_s_b"]],
        params["ncf_t_W"], [b.reshape(1, -1) for b in params["ncf_t_b"]],
        params["dense_s_W"], params["dense_s_b"].reshape(1, 1),
        params["dense_t_W"], params["dense_t_b"].reshape(1, 1),
    )


# speedup vs baseline: 13.9172x; 1.5077x over previous
import functools

import jax
import jax.numpy as jnp
from jax import lax
from jax.experimental import pallas as pl
from jax.experimental.pallas import tpu as pltpu
from jax.experimental.pallas import tpu_sc as plsc

_NC = 2
_NS = 16
_NW = _NC * _NS
_STREAM = 128
_KCH = 4
_CH = _KCH * _STREAM


def _sc_mesh():
    return plsc.VectorSubcoreMesh(core_axis_name="c", subcore_axis_name="s",
                                  num_cores=_NC, num_subcores=_NS)


def _build_msgpass(n_pad, groups):
    rpt = n_pad // _NS
    assert groups % 2 == 0

    @functools.partial(
        pl.kernel,
        out_type=jax.ShapeDtypeStruct((2, n_pad, 16), jnp.float32),
        mesh=_sc_mesh(),
        compiler_params=pltpu.CompilerParams(use_tc_tiling_on_sc=False,
                                            needs_layout_passes=False),
        scratch_types=[
            pltpu.VMEM_SHARED((n_pad, 16), jnp.float32),
            pltpu.VMEM((2, _KCH, 3, _STREAM), jnp.int32),
            pltpu.VMEM((2, _KCH, _STREAM, 16), jnp.float32),
            pltpu.SemaphoreType.DMA,
            pltpu.SemaphoreType.DMA,
            pltpu.SemaphoreType.DMA,
            pltpu.SemaphoreType.DMA,
        ],
        name="sc_msgpass",
    )
    def msgpass(h_a, h_b, zeros_ref, epack, out,
                acc, ibuf, rows, semg0, semg1, sems0, sems1):
        c = lax.axis_index("c")
        s = lax.axis_index("s")

        pltpu.sync_copy(zeros_ref.at[pl.ds(s * rpt, rpt)],
                        acc.at[pl.ds(s * rpt, rpt)])
        plsc.subcore_barrier()

        row0 = s * groups * _KCH

        def fire_gathers(buf, semg):
            for j in range(_KCH):
                @pl.when(c == 0)
                def _fa(j=j):
                    pltpu.async_copy(h_a.at[ibuf.at[buf, j, 0]],
                                     rows.at[buf].at[j], semg)

                @pl.when(c == 1)
                def _fb(j=j):
                    pltpu.async_copy(h_b.at[ibuf.at[buf, j, 0]],
                                     rows.at[buf].at[j], semg)

        def drain_gathers(buf, semg):
            for j in range(_KCH):
                pltpu.make_async_copy(h_a.at[ibuf.at[buf, j, 0]],
                                      rows.at[buf].at[j], semg).wait()

        def fire_scatters(buf, sems):
            for j in range(_KCH):
                pltpu.async_copy(rows.at[buf].at[j],
                                 acc.at[ibuf.at[buf, j, 1]], sems, add=True)

        def drain_scatters(buf, sems):
            for j in range(_KCH):
                pltpu.make_async_copy(rows.at[buf].at[j],
                                      acc.at[ibuf.at[buf, j, 1]], sems).wait()

        def multiply(buf):
            for j in range(_KCH):
                @pl.loop(0, _STREAM // 16)
                def _mul(i16, j=j):
                    base = i16 * 16
                    w16 = plsc.bitcast(ibuf[buf, j, 2, pl.ds(base, 16)],
                                       jnp.float32)
                    for l in range(16):
                        wb = jnp.take_along_axis(
                            w16, jnp.full((16,), l, jnp.int32), axis=0)
                        rows[buf, j, base + l, :] = (
                            rows[buf, j, base + l, :] * wb)

        def phase(g, cur, nxt, semg_c, semg_n, sems_c, sems_n):
            drain_gathers(cur, semg_c)
            multiply(cur)
            fire_scatters(cur, sems_c)

            @pl.when(g > 0)
            def _():
                drain_scatters(nxt, sems_n)

            @pl.when(g + 1 < groups)
            def _():
                r0 = row0 + (g + 1) * _KCH
                pltpu.sync_copy(epack.at[pl.ds(r0, _KCH)], ibuf.at[nxt])
                fire_gathers(nxt, semg_n)

        pltpu.sync_copy(epack.at[pl.ds(row0, _KCH)], ibuf.at[0])
        fire_gathers(0, semg0)

        @pl.loop(0, groups // 2)
        def _pair(t):
            phase(2 * t, 0, 1, semg0, semg1, sems0, sems1)
            phase(2 * t + 1, 1, 0, semg1, semg0, sems1, sems0)

        drain_scatters(1, sems1)

        plsc.subcore_barrier()
        pltpu.sync_copy(acc.at[pl.ds(s * rpt, rpt)],
                        out.at[c].at[pl.ds(s * rpt, rpt)])

    return msgpass


def _build_gather_cat(n_pad, bsz):
    per_tile = bsz // _NW

    @functools.partial(
        pl.kernel,
        out_type=jax.ShapeDtypeStruct((3, bsz, 128), jnp.float32),
        mesh=_sc_mesh(),
        compiler_params=pltpu.CompilerParams(use_tc_tiling_on_sc=False,
                                            needs_layout_passes=False),
        scratch_types=[
            pltpu.VMEM((per_tile,), jnp.int32),
            pltpu.VMEM((8, per_tile, 16), jnp.float32),
            pltpu.VMEM((per_tile, 128), jnp.float32),
            pltpu.SemaphoreType.DMA,
        ],
        name="sc_gather_cat",
    )
    def gather_cat(p0, p1, p2, p3, p4, p5, p6, p7, gidx_flat, out,
                   idxv, tmp, buf, sem):
        c = lax.axis_index("c")
        s = lax.axis_index("s")
        wid = s * _NC + c
        parts = (p0, p1, p2, p3, p4, p5, p6, p7)

        @pl.loop(0, 3)
        def _set(jset):
            pltpu.sync_copy(
                gidx_flat.at[pl.ds((jset * _NW + wid) * per_tile, per_tile)],
                idxv)
            for p in range(8):
                pltpu.async_copy(parts[p].at[idxv], tmp.at[p], sem)
            for p in range(8):
                pltpu.make_async_copy(parts[p].at[idxv], tmp.at[p], sem).wait()
            for p in range(8):
                @pl.loop(0, per_tile)
                def _cp(i, p=p):
                    buf[i, pl.ds(p * 16, 16)] = tmp[p, i, :]
            pltpu.sync_copy(buf, out.at[jset].at[pl.ds(wid * per_tile, per_tile)])

    return gather_cat


def _tc_layer_packed(agg_pk, w, b):
    npk = agg_pk.shape[1]
    eye8 = jnp.eye(8, dtype=jnp.float32)
    waa = jnp.kron(eye8, w[:16, :16])
    wba = jnp.kron(eye8, w[16:, :16])
    wab = jnp.kron(eye8, w[:16, 16:])
    wbb = jnp.kron(eye8, w[16:, 16:])
    bias_a = jnp.tile(b[:16], 8).reshape(1, 128)
    bias_b = jnp.tile(b[16:], 8).reshape(1, 128)

    r = npk // 4
    assert r % 8 == 0

    def body(a_ref, b_ref, waa_r, wba_r, wab_r, wbb_r, ba_r, bb_r,
             oa_ref, ob_ref):
        a = a_ref[0]
        bm = b_ref[0]
        ha = (jnp.dot(a, waa_r[...], preferred_element_type=jnp.float32)
              + jnp.dot(bm, wba_r[...], preferred_element_type=jnp.float32))
        hb = (jnp.dot(a, wab_r[...], preferred_element_type=jnp.float32)
              + jnp.dot(bm, wbb_r[...], preferred_element_type=jnp.float32))
        oa_ref[...] = jnp.maximum(ha + ba_r[...], 0.0)
        ob_ref[...] = jnp.maximum(hb + bb_r[...], 0.0)

    def wspec(i):
        return pl.BlockSpec((128, 128), lambda i: (0, 0))

    ha, hb = pl.pallas_call(
        body,
        grid=(4,),
        in_specs=[
            pl.BlockSpec((1, r, 128), lambda i: (0, i, 0)),
            pl.BlockSpec((1, r, 128), lambda i: (1, i, 0)),
            wspec(0), wspec(1), wspec(2), wspec(3),
            pl.BlockSpec((1, 128), lambda i: (0, 0)),
            pl.BlockSpec((1, 128), lambda i: (0, 0)),
        ],
        out_specs=[
            pl.BlockSpec((r, 128), lambda i: (i, 0)),
            pl.BlockSpec((r, 128), lambda i: (i, 0)),
        ],
        out_shape=[jax.ShapeDtypeStruct((npk, 128), jnp.float32)] * 2,
        name="tc_gnn_layer",
    )(agg_pk, agg_pk, waa, wba, wab, wbb, bias_a, bias_b)
    return ha, hb


def _tc_mlp(ug, sig, tig, ws, bs, wt, bt, dws, dbs, dwt, dbt):
    bsz = ug.shape[0]
    rb = 512
    grid = bsz // rb

    def body(u_ref, s_ref, t_ref,
             ws0, ws1, ws2, bs0, bs1, bs2,
             wt0, wt1, wt2, bt0, bt1, bt2,
             dws_ref, dbs_ref, dwt_ref, dbt_ref, out_ref):
        u = u_ref[...]
        xs = jnp.concatenate([u, s_ref[...]], axis=1)
        xt = jnp.concatenate([u, t_ref[...]], axis=1)
        for wr, br in ((ws0, bs0), (ws1, bs1), (ws2, bs2)):
            xs = jnp.maximum(
                jnp.dot(xs, wr[...], preferred_element_type=jnp.float32) + br[...], 0.0)
        for wr, br in ((wt0, bt0), (wt1, bt1), (wt2, bt2)):
            xt = jnp.maximum(
                jnp.dot(xt, wr[...], preferred_element_type=jnp.float32) + br[...], 0.0)
        ss = jnp.dot(xs, dws_ref[...], preferred_element_type=jnp.float32) + dbs_ref[...]
        st = jnp.dot(xt, dwt_ref[...], preferred_element_type=jnp.float32) + dbt_ref[...]
        out_ref[...] = jnp.concatenate([ss, st], axis=1)

    def wspec(shape):
        return pl.BlockSpec(shape, lambda i: tuple(0 for _ in shape))

    in_specs = [pl.BlockSpec((rb, 128), lambda i: (i, 0))] * 3
    in_specs += [wspec(w.shape) for w in ws]
    in_specs += [wspec(b.shape) for b in bs]
    in_specs += [wspec(w.shape) for w in wt]
    in_specs += [wspec(b.shape) for b in bt]
    in_specs += [wspec(dws.shape), wspec(dbs.shape), wspec(dwt.shape), wspec(dbt.shape)]

    return pl.pallas_call(
        body,
        grid=(grid,),
        in_specs=in_specs,
        out_specs=pl.BlockSpec((rb, 2), lambda i: (i, 0)),
        out_shape=jax.ShapeDtypeStruct((bsz, 2), jnp.float32),
        name="tc_ncf_mlp",
    )(ug, sig, tig, *ws, *bs, *wt, *bt, dws, dbs, dwt, dbt)


def kernel(params, edge_weight, u, si, ti, edge_index):
    user_n = params["user_emb"].shape[0]
    i1_n = params["item_s_emb"].shape[0]
    n_nodes = user_n + i1_n + params["item_t_emb"].shape[0]
    e = edge_index.shape[1]
    bsz = u.shape[0]

    per_sub = -(-e // (_NS * 2 * _CH)) * (2 * _CH)
    e_pad = per_sub * _NS
    groups = per_sub // _CH
    pad = e_pad - e

    src = jnp.concatenate(
        [edge_index[0], jnp.zeros((pad,), edge_index.dtype)]).reshape(-1, _STREAM)
    dst = jnp.concatenate(
        [edge_index[1], jnp.zeros((pad,), edge_index.dtype)]).reshape(-1, _STREAM)
    wbits = lax.bitcast_convert_type(
        jnp.concatenate([edge_weight, jnp.zeros((pad,), edge_weight.dtype)]),
        jnp.int32).reshape(-1, _STREAM)
    epack = jnp.stack([src, dst, wbits], axis=1)

    n_pad = -(-n_nodes // (_NS * 8)) * (_NS * 8)
    npk = n_pad // 8

    ego = jnp.concatenate(
        [params["user_emb"], params["item_s_emb"], params["item_t_emb"]], axis=0)
    ego = jnp.concatenate(
        [ego, jnp.zeros((n_pad - n_nodes, ego.shape[1]), ego.dtype)], axis=0)
    h_pa = ego[:, :16].reshape(npk, 128)
    h_pb = ego[:, 16:].reshape(npk, 128)

    zeros_n16 = jnp.zeros((n_pad, 16), jnp.float32)
    msgpass = _build_msgpass(n_pad, groups)

    parts = [h_pa.reshape(n_pad, 16), h_pb.reshape(n_pad, 16)]
    for k in range(len(params["gnn_W"])):
        agg = msgpass(parts[-2], parts[-1], zeros_n16, epack)
        h_pa, h_pb = _tc_layer_packed(agg.reshape(2, npk, 128),
                                      params["gnn_W"][k], params["gnn_b"][k])
        parts += [h_pa.reshape(n_pad, 16), h_pb.reshape(n_pad, 16)]

    gidx_flat = jnp.stack([u, si + user_n, ti + user_n + i1_n]).reshape(-1)
    gath = _build_gather_cat(n_pad, bsz)(*parts, gidx_flat)

    return _tc_mlp(
        gath[0], gath[1], gath[2],
        params["ncf_s_W"], [b.reshape(1, -1) for b in params["ncf_s_b"]],
        params["ncf_t_W"], [b.reshape(1, -1) for b in params["ncf_t_b"]],
        params["dense_s_W"], params["dense_s_b"].reshape(1, 1),
        params["dense_t_W"], params["dense_t_b"].reshape(1, 1),
    )


# speedup vs baseline: 17.4354x; 1.2528x over previous
import functools

import jax
import jax.numpy as jnp
from jax import lax
from jax.experimental import pallas as pl
from jax.experimental.pallas import tpu as pltpu
from jax.experimental.pallas import tpu_sc as plsc

_NC = 2
_NS = 16
_NW = _NC * _NS
_STREAM = 128
_KCH = 4
_CH = _KCH * _STREAM


def _sc_mesh():
    return plsc.VectorSubcoreMesh(core_axis_name="c", subcore_axis_name="s",
                                  num_cores=_NC, num_subcores=_NS)


def _build_msgpass(n_pad, groups):
    rpt = n_pad // _NS
    assert groups % 2 == 0

    @functools.partial(
        pl.kernel,
        out_type=jax.ShapeDtypeStruct((2, n_pad, 16), jnp.float32),
        mesh=_sc_mesh(),
        compiler_params=pltpu.CompilerParams(use_tc_tiling_on_sc=False,
                                            needs_layout_passes=False),
        scratch_types=[
            pltpu.VMEM_SHARED((n_pad, 16), jnp.float32),
            pltpu.VMEM((2, _KCH, 2, _STREAM), jnp.int32),
            pltpu.VMEM((2, _KCH, _STREAM), jnp.float32),
            pltpu.VMEM((2, _KCH, _STREAM, 16), jnp.float32),
            pltpu.SemaphoreType.DMA,
            pltpu.SemaphoreType.DMA,
            pltpu.SemaphoreType.DMA,
            pltpu.SemaphoreType.DMA,
            pltpu.SemaphoreType.DMA,
            pltpu.SemaphoreType.DMA,
        ],
        name="sc_msgpass",
    )
    def msgpass(h_a, h_b, zeros_ref, ep2, wp, out,
                acc, ibuf, wbuf, rows, semg0, semg1, sems0, sems1,
                semi0, semi1):
        c = lax.axis_index("c")
        s = lax.axis_index("s")

        pltpu.sync_copy(zeros_ref.at[pl.ds(s * rpt, rpt)],
                        acc.at[pl.ds(s * rpt, rpt)])
        plsc.subcore_barrier()

        row0 = s * groups * _KCH

        def fire_idx(g, buf, semi):
            r0 = row0 + g * _KCH
            pltpu.async_copy(ep2.at[pl.ds(r0, _KCH)], ibuf.at[buf], semi)
            pltpu.async_copy(wp.at[pl.ds(r0, _KCH)], wbuf.at[buf], semi)

        def drain_idx(g, buf, semi):
            r0 = row0 + g * _KCH
            pltpu.make_async_copy(ep2.at[pl.ds(r0, _KCH)], ibuf.at[buf],
                                  semi).wait()
            pltpu.make_async_copy(wp.at[pl.ds(r0, _KCH)], wbuf.at[buf],
                                  semi).wait()

        def fire_gathers(buf, semg):
            for j in range(_KCH):
                @pl.when(c == 0)
                def _fa(j=j):
                    pltpu.async_copy(h_a.at[ibuf.at[buf, j, 0]],
                                     rows.at[buf].at[j], semg)

                @pl.when(c == 1)
                def _fb(j=j):
                    pltpu.async_copy(h_b.at[ibuf.at[buf, j, 0]],
                                     rows.at[buf].at[j], semg)

        def drain_gathers(buf, semg):
            for j in range(_KCH):
                pltpu.make_async_copy(h_a.at[ibuf.at[buf, j, 0]],
                                      rows.at[buf].at[j], semg).wait()

        def fire_scatters(buf, sems):
            for j in range(_KCH):
                pltpu.async_copy(rows.at[buf].at[j],
                                 acc.at[ibuf.at[buf, j, 1]], sems, add=True)

        def drain_scatters(buf, sems):
            for j in range(_KCH):
                pltpu.make_async_copy(rows.at[buf].at[j],
                                      acc.at[ibuf.at[buf, j, 1]], sems).wait()

        def multiply(buf):
            for j in range(_KCH):
                @pl.loop(0, _STREAM // 16, unroll=2)
                def _mul(i16, j=j):
                    base = i16 * 16
                    w16 = wbuf[buf, j, pl.ds(base, 16)]
                    for l in range(16):
                        wb = jnp.take_along_axis(
                            w16, jnp.full((16,), l, jnp.int32), axis=0)
                        rows[buf, j, base + l, :] = (
                            rows[buf, j, base + l, :] * wb)

        def phase(g, cur, nxt, semg_c, semg_n, sems_c, sems_n, semi_n):
            drain_gathers(cur, semg_c)

            @pl.when(g > 0)
            def _():
                drain_scatters(nxt, sems_n)

            @pl.when(g + 1 < groups)
            def _():
                fire_idx(g + 1, nxt, semi_n)

            multiply(cur)
            fire_scatters(cur, sems_c)

            @pl.when(g + 1 < groups)
            def _():
                drain_idx(g + 1, nxt, semi_n)
                fire_gathers(nxt, semg_n)

        pltpu.sync_copy(ep2.at[pl.ds(row0, _KCH)], ibuf.at[0])
        pltpu.sync_copy(wp.at[pl.ds(row0, _KCH)], wbuf.at[0])
        fire_gathers(0, semg0)

        @pl.loop(0, groups // 2)
        def _pair(t):
            phase(2 * t, 0, 1, semg0, semg1, sems0, sems1, semi1)
            phase(2 * t + 1, 1, 0, semg1, semg0, sems1, sems0, semi0)

        drain_scatters(1, sems1)

        plsc.subcore_barrier()
        pltpu.sync_copy(acc.at[pl.ds(s * rpt, rpt)],
                        out.at[c].at[pl.ds(s * rpt, rpt)])

    return msgpass


def _build_gather_cat(n_pad, bsz):
    per_tile = bsz // _NW

    @functools.partial(
        pl.kernel,
        out_type=jax.ShapeDtypeStruct((3, bsz, 128), jnp.float32),
        mesh=_sc_mesh(),
        compiler_params=pltpu.CompilerParams(use_tc_tiling_on_sc=False,
                                            needs_layout_passes=False),
        scratch_types=[
            pltpu.VMEM((per_tile,), jnp.int32),
            pltpu.VMEM((8, per_tile, 16), jnp.float32),
            pltpu.VMEM((per_tile, 128), jnp.float32),
            pltpu.SemaphoreType.DMA,
        ],
        name="sc_gather_cat",
    )
    def gather_cat(p0, p1, p2, p3, p4, p5, p6, p7, gidx_flat, out,
                   idxv, tmp, buf, sem):
        c = lax.axis_index("c")
        s = lax.axis_index("s")
        wid = s * _NC + c
        parts = (p0, p1, p2, p3, p4, p5, p6, p7)

        @pl.loop(0, 3)
        def _set(jset):
            pltpu.sync_copy(
                gidx_flat.at[pl.ds((jset * _NW + wid) * per_tile, per_tile)],
                idxv)
            for p in range(8):
                pltpu.async_copy(parts[p].at[idxv], tmp.at[p], sem)
            for p in range(8):
                pltpu.make_async_copy(parts[p].at[idxv], tmp.at[p], sem).wait()
            for p in range(8):
                @pl.loop(0, per_tile)
                def _cp(i, p=p):
                    buf[i, pl.ds(p * 16, 16)] = tmp[p, i, :]
            pltpu.sync_copy(buf, out.at[jset].at[pl.ds(wid * per_tile, per_tile)])

    return gather_cat


def _tc_layer_packed(agg_pk, w, b):
    npk = agg_pk.shape[1]
    eye8 = jnp.eye(8, dtype=jnp.float32)
    waa = jnp.kron(eye8, w[:16, :16])
    wba = jnp.kron(eye8, w[16:, :16])
    wab = jnp.kron(eye8, w[:16, 16:])
    wbb = jnp.kron(eye8, w[16:, 16:])
    bias_a = jnp.tile(b[:16], 8).reshape(1, 128)
    bias_b = jnp.tile(b[16:], 8).reshape(1, 128)

    r = npk // 4
    assert r % 8 == 0

    def body(a_ref, b_ref, waa_r, wba_r, wab_r, wbb_r, ba_r, bb_r,
             oa_ref, ob_ref):
        a = a_ref[0]
        bm = b_ref[0]
        ha = (jnp.dot(a, waa_r[...], preferred_element_type=jnp.float32)
              + jnp.dot(bm, wba_r[...], preferred_element_type=jnp.float32))
        hb = (jnp.dot(a, wab_r[...], preferred_element_type=jnp.float32)
              + jnp.dot(bm, wbb_r[...], preferred_element_type=jnp.float32))
        oa_ref[...] = jnp.maximum(ha + ba_r[...], 0.0)
        ob_ref[...] = jnp.maximum(hb + bb_r[...], 0.0)

    def wspec(i):
        return pl.BlockSpec((128, 128), lambda i: (0, 0))

    ha, hb = pl.pallas_call(
        body,
        grid=(4,),
        in_specs=[
            pl.BlockSpec((1, r, 128), lambda i: (0, i, 0)),
            pl.BlockSpec((1, r, 128), lambda i: (1, i, 0)),
            wspec(0), wspec(1), wspec(2), wspec(3),
            pl.BlockSpec((1, 128), lambda i: (0, 0)),
            pl.BlockSpec((1, 128), lambda i: (0, 0)),
        ],
        out_specs=[
            pl.BlockSpec((r, 128), lambda i: (i, 0)),
            pl.BlockSpec((r, 128), lambda i: (i, 0)),
        ],
        out_shape=[jax.ShapeDtypeStruct((npk, 128), jnp.float32)] * 2,
        name="tc_gnn_layer",
    )(agg_pk, agg_pk, waa, wba, wab, wbb, bias_a, bias_b)
    return ha, hb


def _tc_mlp(ug, sig, tig, ws, bs, wt, bt, dws, dbs, dwt, dbt):
    bsz = ug.shape[0]
    rb = 512
    grid = bsz // rb

    def body(u_ref, s_ref, t_ref,
             ws0, ws1, ws2, bs0, bs1, bs2,
             wt0, wt1, wt2, bt0, bt1, bt2,
             dws_ref, dbs_ref, dwt_ref, dbt_ref, out_ref):
        u = u_ref[...]
        xs = jnp.concatenate([u, s_ref[...]], axis=1)
        xt = jnp.concatenate([u, t_ref[...]], axis=1)
        for wr, br in ((ws0, bs0), (ws1, bs1), (ws2, bs2)):
            xs = jnp.maximum(
                jnp.dot(xs, wr[...], preferred_element_type=jnp.float32) + br[...], 0.0)
        for wr, br in ((wt0, bt0), (wt1, bt1), (wt2, bt2)):
            xt = jnp.maximum(
                jnp.dot(xt, wr[...], preferred_element_type=jnp.float32) + br[...], 0.0)
        ss = jnp.dot(xs, dws_ref[...], preferred_element_type=jnp.float32) + dbs_ref[...]
        st = jnp.dot(xt, dwt_ref[...], preferred_element_type=jnp.float32) + dbt_ref[...]
        out_ref[...] = jnp.concatenate([ss, st], axis=1)

    def wspec(shape):
        return pl.BlockSpec(shape, lambda i: tuple(0 for _ in shape))

    in_specs = [pl.BlockSpec((rb, 128), lambda i: (i, 0))] * 3
    in_specs += [wspec(w.shape) for w in ws]
    in_specs += [wspec(b.shape) for b in bs]
    in_specs += [wspec(w.shape) for w in wt]
    in_specs += [wspec(b.shape) for b in bt]
    in_specs += [wspec(dws.shape), wspec(dbs.shape), wspec(dwt.shape), wspec(dbt.shape)]

    return pl.pallas_call(
        body,
        grid=(grid,),
        in_specs=in_specs,
        out_specs=pl.BlockSpec((rb, 2), lambda i: (i, 0)),
        out_shape=jax.ShapeDtypeStruct((bsz, 2), jnp.float32),
        name="tc_ncf_mlp",
    )(ug, sig, tig, *ws, *bs, *wt, *bt, dws, dbs, dwt, dbt)


def kernel(params, edge_weight, u, si, ti, edge_index):
    user_n = params["user_emb"].shape[0]
    i1_n = params["item_s_emb"].shape[0]
    n_nodes = user_n + i1_n + params["item_t_emb"].shape[0]
    e = edge_index.shape[1]
    bsz = u.shape[0]

    per_sub = -(-e // (_NS * 2 * _CH)) * (2 * _CH)
    e_pad = per_sub * _NS
    groups = per_sub // _CH
    pad = e_pad - e

    ep2 = jnp.swapaxes(edge_index.reshape(2, -1, _STREAM), 0, 1)
    ep2 = jnp.concatenate(
        [ep2, jnp.zeros((pad // _STREAM, 2, _STREAM), edge_index.dtype)], axis=0)
    wp = jnp.concatenate(
        [edge_weight, jnp.zeros((pad,), edge_weight.dtype)]).reshape(-1, _STREAM)

    n_pad = -(-n_nodes // (_NS * 8)) * (_NS * 8)
    npk = n_pad // 8

    ego = jnp.concatenate(
        [params["user_emb"], params["item_s_emb"], params["item_t_emb"]], axis=0)
    ego = jnp.concatenate(
        [ego, jnp.zeros((n_pad - n_nodes, ego.shape[1]), ego.dtype)], axis=0)
    h_pa = ego[:, :16].reshape(npk, 128)
    h_pb = ego[:, 16:].reshape(npk, 128)

    zeros_n16 = jnp.zeros((n_pad, 16), jnp.float32)
    msgpass = _build_msgpass(n_pad, groups)

    parts = [h_pa.reshape(n_pad, 16), h_pb.reshape(n_pad, 16)]
    for k in range(len(params["gnn_W"])):
        agg = msgpass(parts[-2], parts[-1], zeros_n16, ep2, wp)
        h_pa, h_pb = _tc_layer_packed(agg.reshape(2, npk, 128),
                                      params["gnn_W"][k], params["gnn_b"][k])
        parts += [h_pa.reshape(n_pad, 16), h_pb.reshape(n_pad, 16)]

    gidx_flat = jnp.stack([u, si + user_n, ti + user_n + i1_n]).reshape(-1)
    gath = _build_gather_cat(n_pad, bsz)(*parts, gidx_flat)

    return _tc_mlp(
        gath[0], gath[1], gath[2],
        params["ncf_s_W"], [b.reshape(1, -1) for b in params["ncf_s_b"]],
        params["ncf_t_W"], [b.reshape(1, -1) for b in params["ncf_t_b"]],
        params["dense_s_W"], params["dense_s_b"].reshape(1, 1),
        params["dense_t_W"], params["dense_t_b"].reshape(1, 1),
    )


# speedup vs baseline: 19.5509x; 1.1213x over previous
import functools

import jax
import jax.numpy as jnp
from jax import lax
from jax.experimental import pallas as pl
from jax.experimental.pallas import tpu as pltpu
from jax.experimental.pallas import tpu_sc as plsc

_NC = 2
_NS = 16
_NW = _NC * _NS
_STREAM = 128
_KCH = 5
_CH = _KCH * _STREAM


def _sc_mesh():
    return plsc.VectorSubcoreMesh(core_axis_name="c", subcore_axis_name="s",
                                  num_cores=_NC, num_subcores=_NS)


def _build_msgpass(n_pad, groups, rows_real):
    rpt = n_pad // _NS
    assert groups % 2 == 0

    @functools.partial(
        pl.kernel,
        out_type=jax.ShapeDtypeStruct((2, n_pad, 16), jnp.float32),
        mesh=_sc_mesh(),
        compiler_params=pltpu.CompilerParams(use_tc_tiling_on_sc=False,
                                            needs_layout_passes=False),
        scratch_types=[
            pltpu.VMEM_SHARED((n_pad, 16), jnp.float32),
            pltpu.VMEM((2, _KCH, 2, _STREAM), jnp.int32),
            pltpu.VMEM((2, _KCH, _STREAM), jnp.float32),
            pltpu.VMEM((2, _KCH, _STREAM, 16), jnp.float32),
            pltpu.SemaphoreType.DMA,
            pltpu.SemaphoreType.DMA,
            pltpu.SemaphoreType.DMA,
            pltpu.SemaphoreType.DMA,
            pltpu.SemaphoreType.DMA,
            pltpu.SemaphoreType.DMA,
        ],
        name="sc_msgpass",
    )
    def msgpass(h_a, h_b, zeros_ref, ep2, wp, out,
                acc, ibuf, wbuf, rows, semg0, semg1, sems0, sems1,
                semi0, semi1):
        c = lax.axis_index("c")
        s = lax.axis_index("s")

        pltpu.sync_copy(zeros_ref.at[pl.ds(s * rpt, rpt)],
                        acc.at[pl.ds(s * rpt, rpt)])
        plsc.subcore_barrier()

        row0 = s * groups * _KCH

        def fire_idx(g, buf, semi):
            r0 = row0 + g * _KCH
            re = jnp.minimum(r0, rows_real - _KCH)
            pltpu.async_copy(ep2.at[pl.ds(re, _KCH)], ibuf.at[buf], semi)
            pltpu.async_copy(wp.at[pl.ds(r0, _KCH)], wbuf.at[buf], semi)

        def drain_idx(g, buf, semi):
            r0 = row0 + g * _KCH
            re = jnp.minimum(r0, rows_real - _KCH)
            pltpu.make_async_copy(ep2.at[pl.ds(re, _KCH)], ibuf.at[buf],
                                  semi).wait()
            pltpu.make_async_copy(wp.at[pl.ds(r0, _KCH)], wbuf.at[buf],
                                  semi).wait()

        def fire_gathers(buf, semg):
            for j in range(_KCH):
                @pl.when(c == 0)
                def _fa(j=j):
                    pltpu.async_copy(h_a.at[ibuf.at[buf, j, 0]],
                                     rows.at[buf].at[j], semg)

                @pl.when(c == 1)
                def _fb(j=j):
                    pltpu.async_copy(h_b.at[ibuf.at[buf, j, 0]],
                                     rows.at[buf].at[j], semg)

        def drain_gathers(buf, semg):
            for j in range(_KCH):
                pltpu.make_async_copy(h_a.at[ibuf.at[buf, j, 0]],
                                      rows.at[buf].at[j], semg).wait()

        def fire_scatters(buf, sems):
            for j in range(_KCH):
                pltpu.async_copy(rows.at[buf].at[j],
                                 acc.at[ibuf.at[buf, j, 1]], sems, add=True)

        def drain_scatters(buf, sems):
            for j in range(_KCH):
                pltpu.make_async_copy(rows.at[buf].at[j],
                                      acc.at[ibuf.at[buf, j, 1]], sems).wait()

        def multiply(buf):
            for j in range(_KCH):
                @pl.loop(0, _STREAM // 16, unroll=2)
                def _mul(i16, j=j):
                    base = i16 * 16
                    w16 = wbuf[buf, j, pl.ds(base, 16)]
                    for l in range(16):
                        wb = jnp.take_along_axis(
                            w16, jnp.full((16,), l, jnp.int32), axis=0)
                        rows[buf, j, base + l, :] = (
                            rows[buf, j, base + l, :] * wb)

        def phase(g, cur, nxt, semg_c, semg_n, sems_c, sems_n, semi_n):
            drain_gathers(cur, semg_c)

            @pl.when(g > 0)
            def _():
                drain_scatters(nxt, sems_n)

            @pl.when(g + 1 < groups)
            def _():
                fire_idx(g + 1, nxt, semi_n)

            multiply(cur)
            fire_scatters(cur, sems_c)

            @pl.when(g + 1 < groups)
            def _():
                drain_idx(g + 1, nxt, semi_n)
                fire_gathers(nxt, semg_n)

        pltpu.sync_copy(ep2.at[pl.ds(jnp.minimum(row0, rows_real - _KCH), _KCH)],
                        ibuf.at[0])
        pltpu.sync_copy(wp.at[pl.ds(row0, _KCH)], wbuf.at[0])
        fire_gathers(0, semg0)

        @pl.loop(0, groups // 2)
        def _pair(t):
            phase(2 * t, 0, 1, semg0, semg1, sems0, sems1, semi1)
            phase(2 * t + 1, 1, 0, semg1, semg0, sems1, sems0, semi0)

        drain_scatters(1, sems1)

        plsc.subcore_barrier()
        pltpu.sync_copy(acc.at[pl.ds(s * rpt, rpt)],
                        out.at[c].at[pl.ds(s * rpt, rpt)])

    return msgpass


def _build_gather_cat(n_pad, bsz):
    per_tile = bsz // _NW

    @functools.partial(
        pl.kernel,
        out_type=jax.ShapeDtypeStruct((3, bsz, 128), jnp.float32),
        mesh=_sc_mesh(),
        compiler_params=pltpu.CompilerParams(use_tc_tiling_on_sc=False,
                                            needs_layout_passes=False),
        scratch_types=[
            pltpu.VMEM((per_tile,), jnp.int32),
            pltpu.VMEM((8, per_tile, 16), jnp.float32),
            pltpu.VMEM((per_tile, 128), jnp.float32),
            pltpu.SemaphoreType.DMA,
        ],
        name="sc_gather_cat",
    )
    def gather_cat(p0, p1, p2, p3, p4, p5, p6, p7, gidx_flat, out,
                   idxv, tmp, buf, sem):
        c = lax.axis_index("c")
        s = lax.axis_index("s")
        wid = s * _NC + c
        parts = (p0, p1, p2, p3, p4, p5, p6, p7)

        @pl.loop(0, 3)
        def _set(jset):
            pltpu.sync_copy(
                gidx_flat.at[pl.ds((jset * _NW + wid) * per_tile, per_tile)],
                idxv)
            for p in range(8):
                pltpu.async_copy(parts[p].at[idxv], tmp.at[p], sem)
            for p in range(8):
                pltpu.make_async_copy(parts[p].at[idxv], tmp.at[p], sem).wait()
            for p in range(8):
                @pl.loop(0, per_tile)
                def _cp(i, p=p):
                    buf[i, pl.ds(p * 16, 16)] = tmp[p, i, :]
            pltpu.sync_copy(buf, out.at[jset].at[pl.ds(wid * per_tile, per_tile)])

    return gather_cat


def _tc_layer_packed(agg_pk, w, b):
    npk = agg_pk.shape[1]
    eye8 = jnp.eye(8, dtype=jnp.float32)
    waa = jnp.kron(eye8, w[:16, :16])
    wba = jnp.kron(eye8, w[16:, :16])
    wab = jnp.kron(eye8, w[:16, 16:])
    wbb = jnp.kron(eye8, w[16:, 16:])
    bias_a = jnp.tile(b[:16], 8).reshape(1, 128)
    bias_b = jnp.tile(b[16:], 8).reshape(1, 128)

    r = npk // 4
    assert r % 8 == 0

    def body(a_ref, b_ref, waa_r, wba_r, wab_r, wbb_r, ba_r, bb_r,
             oa_ref, ob_ref):
        a = a_ref[0]
        bm = b_ref[0]
        ha = (jnp.dot(a, waa_r[...], preferred_element_type=jnp.float32)
              + jnp.dot(bm, wba_r[...], preferred_element_type=jnp.float32))
        hb = (jnp.dot(a, wab_r[...], preferred_element_type=jnp.float32)
              + jnp.dot(bm, wbb_r[...], preferred_element_type=jnp.float32))
        oa_ref[...] = jnp.maximum(ha + ba_r[...], 0.0)
        ob_ref[...] = jnp.maximum(hb + bb_r[...], 0.0)

    def wspec(i):
        return pl.BlockSpec((128, 128), lambda i: (0, 0))

    ha, hb = pl.pallas_call(
        body,
        grid=(4,),
        in_specs=[
            pl.BlockSpec((1, r, 128), lambda i: (0, i, 0)),
            pl.BlockSpec((1, r, 128), lambda i: (1, i, 0)),
            wspec(0), wspec(1), wspec(2), wspec(3),
            pl.BlockSpec((1, 128), lambda i: (0, 0)),
            pl.BlockSpec((1, 128), lambda i: (0, 0)),
        ],
        out_specs=[
            pl.BlockSpec((r, 128), lambda i: (i, 0)),
            pl.BlockSpec((r, 128), lambda i: (i, 0)),
        ],
        out_shape=[jax.ShapeDtypeStruct((npk, 128), jnp.float32)] * 2,
        name="tc_gnn_layer",
    )(agg_pk, agg_pk, waa, wba, wab, wbb, bias_a, bias_b)
    return ha, hb


def _tc_mlp(ug, sig, tig, ws, bs, wt, bt, dws, dbs, dwt, dbt):
    bsz = ug.shape[0]
    rb = 512
    grid = bsz // rb

    def body(u_ref, s_ref, t_ref,
             ws0, ws1, ws2, bs0, bs1, bs2,
             wt0, wt1, wt2, bt0, bt1, bt2,
             dws_ref, dbs_ref, dwt_ref, dbt_ref, out_ref):
        u = u_ref[...]
        xs = jnp.concatenate([u, s_ref[...]], axis=1)
        xt = jnp.concatenate([u, t_ref[...]], axis=1)
        for wr, br in ((ws0, bs0), (ws1, bs1), (ws2, bs2)):
            xs = jnp.maximum(
                jnp.dot(xs, wr[...], preferred_element_type=jnp.float32) + br[...], 0.0)
        for wr, br in ((wt0, bt0), (wt1, bt1), (wt2, bt2)):
            xt = jnp.maximum(
                jnp.dot(xt, wr[...], preferred_element_type=jnp.float32) + br[...], 0.0)
        ss = jnp.dot(xs, dws_ref[...], preferred_element_type=jnp.float32) + dbs_ref[...]
        st = jnp.dot(xt, dwt_ref[...], preferred_element_type=jnp.float32) + dbt_ref[...]
        out_ref[...] = jnp.concatenate([ss, st], axis=1)

    def wspec(shape):
        return pl.BlockSpec(shape, lambda i: tuple(0 for _ in shape))

    in_specs = [pl.BlockSpec((rb, 128), lambda i: (i, 0))] * 3
    in_specs += [wspec(w.shape) for w in ws]
    in_specs += [wspec(b.shape) for b in bs]
    in_specs += [wspec(w.shape) for w in wt]
    in_specs += [wspec(b.shape) for b in bt]
    in_specs += [wspec(dws.shape), wspec(dbs.shape), wspec(dwt.shape), wspec(dbt.shape)]

    return pl.pallas_call(
        body,
        grid=(grid,),
        in_specs=in_specs,
        out_specs=pl.BlockSpec((rb, 2), lambda i: (i, 0)),
        out_shape=jax.ShapeDtypeStruct((bsz, 2), jnp.float32),
        name="tc_ncf_mlp",
    )(ug, sig, tig, *ws, *bs, *wt, *bt, dws, dbs, dwt, dbt)


def kernel(params, edge_weight, u, si, ti, edge_index):
    user_n = params["user_emb"].shape[0]
    i1_n = params["item_s_emb"].shape[0]
    n_nodes = user_n + i1_n + params["item_t_emb"].shape[0]
    e = edge_index.shape[1]
    bsz = u.shape[0]

    rows_real = e // _STREAM
    rpt_e = -(-rows_real // _NS)
    rpt_e = -(-rpt_e // (2 * _KCH)) * (2 * _KCH)
    groups = rpt_e // _KCH
    ep2 = jnp.swapaxes(edge_index.reshape(2, -1, _STREAM), 0, 1)
    wp = jnp.concatenate(
        [edge_weight,
         jnp.zeros((rpt_e * _NS * _STREAM - e,), edge_weight.dtype)]
    ).reshape(-1, _STREAM)

    n_pad = -(-n_nodes // (_NS * 8)) * (_NS * 8)
    npk = n_pad // 8

    ego = jnp.concatenate(
        [params["user_emb"], params["item_s_emb"], params["item_t_emb"],
         jnp.zeros((n_pad - n_nodes, 32), jnp.float32)], axis=0)
    ego8 = ego.reshape(npk, 8, 32)
    h_pa = ego8[:, :, :16].reshape(npk, 128)
    h_pb = ego8[:, :, 16:].reshape(npk, 128)

    zeros_n16 = jnp.zeros((n_pad, 16), jnp.float32)
    msgpass = _build_msgpass(n_pad, groups, rows_real)

    parts = [h_pa.reshape(n_pad, 16), h_pb.reshape(n_pad, 16)]
    for k in range(len(params["gnn_W"])):
        agg = msgpass(parts[-2], parts[-1], zeros_n16, ep2, wp)
        h_pa, h_pb = _tc_layer_packed(agg.reshape(2, npk, 128),
                                      params["gnn_W"][k], params["gnn_b"][k])
        parts += [h_pa.reshape(n_pad, 16), h_pb.reshape(n_pad, 16)]

    gidx_flat = jnp.stack([u, si + user_n, ti + user_n + i1_n]).reshape(-1)
    gath = _build_gather_cat(n_pad, bsz)(*parts, gidx_flat)

    return _tc_mlp(
        gath[0], gath[1], gath[2],
        params["ncf_s_W"], [b.reshape(1, -1) for b in params["ncf_s_b"]],
        params["ncf_t_W"], [b.reshape(1, -1) for b in params["ncf_t_b"]],
        params["dense_s_W"], params["dense_s_b"].reshape(1, 1),
        params["dense_t_W"], params["dense_t_b"].reshape(1, 1),
    )


# speedup vs baseline: 24.9277x; 1.2750x over previous
import functools

import jax
import jax.numpy as jnp
from jax import lax
from jax.experimental import pallas as pl
from jax.experimental.pallas import tpu as pltpu
from jax.experimental.pallas import tpu_sc as plsc

_NC = 2
_NS = 16
_NW = _NC * _NS
_STREAM = 128
_KCH = 5
_CH = _KCH * _STREAM


def _sc_mesh():
    return plsc.VectorSubcoreMesh(core_axis_name="c", subcore_axis_name="s",
                                  num_cores=_NC, num_subcores=_NS)


def _build_msgpass(n_pad, groups, rows_real):
    rpt = n_pad // _NS
    assert groups % 2 == 0

    @functools.partial(
        pl.kernel,
        out_type=jax.ShapeDtypeStruct((2, n_pad, 16), jnp.float32),
        mesh=_sc_mesh(),
        compiler_params=pltpu.CompilerParams(use_tc_tiling_on_sc=False,
                                            needs_layout_passes=False),
        scratch_types=[
            pltpu.VMEM_SHARED((n_pad, 16), jnp.float32),
            pltpu.VMEM((2, _KCH, 2, _STREAM), jnp.int32),
            pltpu.VMEM((2, _KCH, _STREAM), jnp.int32),
            pltpu.VMEM((2, _KCH, _STREAM), jnp.float32),
            pltpu.VMEM((2, _KCH, _STREAM, 16), jnp.float32),
            pltpu.SemaphoreType.DMA,
            pltpu.SemaphoreType.DMA,
            pltpu.SemaphoreType.DMA,
            pltpu.SemaphoreType.DMA,
            pltpu.SemaphoreType.DMA,
            pltpu.SemaphoreType.DMA,
        ],
        name="sc_msgpass",
    )
    def msgpass(h_a, h_b, zeros_ref, ep2, wp, out,
                acc, ibuf, sbuf, wbuf, rows, semg0, semg1, sems0, sems1,
                semi0, semi1):
        c = lax.axis_index("c")
        s = lax.axis_index("s")

        pltpu.sync_copy(zeros_ref.at[pl.ds(s * rpt, rpt)],
                        acc.at[pl.ds(s * rpt, rpt)])
        plsc.subcore_barrier()

        row0 = s * groups * _KCH

        def fire_idx(g, buf, semi):
            r0 = row0 + g * _KCH
            re = jnp.minimum(r0, rows_real - _KCH)
            pltpu.async_copy(ep2.at[pl.ds(re, _KCH)], ibuf.at[buf], semi)
            pltpu.async_copy(wp.at[pl.ds(r0, _KCH)], wbuf.at[buf], semi)

        def drain_idx(g, buf, semi):
            r0 = row0 + g * _KCH
            re = jnp.minimum(r0, rows_real - _KCH)
            pltpu.make_async_copy(ep2.at[pl.ds(re, _KCH)], ibuf.at[buf],
                                  semi).wait()
            pltpu.make_async_copy(wp.at[pl.ds(r0, _KCH)], wbuf.at[buf],
                                  semi).wait()

        def fire_gathers(buf, semg):
            for j in range(_KCH):
                @pl.when(c == 0)
                def _fa(j=j):
                    pltpu.async_copy(h_a.at[ibuf.at[buf, j, 0]],
                                     rows.at[buf].at[j], semg)

                @pl.when(c == 1)
                def _fb(j=j):
                    pltpu.async_copy(h_b.at[ibuf.at[buf, j, 0]],
                                     rows.at[buf].at[j], semg)

        def drain_gathers(buf, semg):
            for j in range(_KCH):
                pltpu.make_async_copy(h_a.at[ibuf.at[buf, j, 0]],
                                      rows.at[buf].at[j], semg).wait()

        def copy_dst_idx(buf):
            for j in range(_KCH):
                for i16 in range(_STREAM // 16):
                    sbuf[buf, j, pl.ds(i16 * 16, 16)] = (
                        ibuf[buf, j, 1, pl.ds(i16 * 16, 16)])

        def fire_scatters(buf, sems):
            for j in range(_KCH):
                pltpu.async_copy(rows.at[buf].at[j],
                                 acc.at[sbuf.at[buf, j]], sems, add=True)

        def drain_scatters(buf, sems):
            for j in range(_KCH):
                pltpu.make_async_copy(rows.at[buf].at[j],
                                      acc.at[sbuf.at[buf, j]], sems).wait()

        def multiply(buf):
            for j in range(_KCH):
                @pl.loop(0, _STREAM // 16, unroll=4)
                def _mul(i16, j=j):
                    base = i16 * 16
                    w16 = wbuf[buf, j, pl.ds(base, 16)]
                    for l in range(16):
                        wb = jnp.take_along_axis(
                            w16, jnp.full((16,), l, jnp.int32), axis=0)
                        rows[buf, j, base + l, :] = (
                            rows[buf, j, base + l, :] * wb)

        def phase(g, cur, nxt, semg_c, semg_n, sems_c, sems_n,
                  semi_c, semi_n):
            drain_gathers(cur, semg_c)

            @pl.when(g > 0)
            def _():
                drain_scatters(nxt, sems_n)

            @pl.when(g + 1 < groups)
            def _():
                drain_idx(g + 1, nxt, semi_n)
                fire_gathers(nxt, semg_n)

            multiply(cur)
            copy_dst_idx(cur)
            fire_scatters(cur, sems_c)

            @pl.when(g + 2 < groups)
            def _():
                fire_idx(g + 2, cur, semi_c)

        pltpu.sync_copy(ep2.at[pl.ds(jnp.minimum(row0, rows_real - _KCH), _KCH)],
                        ibuf.at[0])
        pltpu.sync_copy(wp.at[pl.ds(row0, _KCH)], wbuf.at[0])
        fire_gathers(0, semg0)
        fire_idx(1, 1, semi1)

        @pl.loop(0, groups // 2)
        def _pair(t):
            phase(2 * t, 0, 1, semg0, semg1, sems0, sems1, semi0, semi1)
            phase(2 * t + 1, 1, 0, semg1, semg0, sems1, sems0, semi1, semi0)

        drain_scatters(1, sems1)

        plsc.subcore_barrier()
        pltpu.sync_copy(acc.at[pl.ds(s * rpt, rpt)],
                        out.at[c].at[pl.ds(s * rpt, rpt)])

    return msgpass


def _build_gather_cat(n_pad, bsz):
    per_tile = bsz // _NW

    @functools.partial(
        pl.kernel,
        out_type=jax.ShapeDtypeStruct((3, bsz, 128), jnp.float32),
        mesh=_sc_mesh(),
        compiler_params=pltpu.CompilerParams(use_tc_tiling_on_sc=False,
                                            needs_layout_passes=False),
        scratch_types=[
            pltpu.VMEM((per_tile,), jnp.int32),
            pltpu.VMEM((8, per_tile, 16), jnp.float32),
            pltpu.VMEM((per_tile, 128), jnp.float32),
            pltpu.SemaphoreType.DMA,
        ],
        name="sc_gather_cat",
    )
    def gather_cat(p0, p1, p2, p3, p4, p5, p6, p7, gidx_flat, out,
                   idxv, tmp, buf, sem):
        c = lax.axis_index("c")
        s = lax.axis_index("s")
        wid = s * _NC + c
        parts = (p0, p1, p2, p3, p4, p5, p6, p7)

        @pl.loop(0, 3)
        def _set(jset):
            pltpu.sync_copy(
                gidx_flat.at[pl.ds((jset * _NW + wid) * per_tile, per_tile)],
                idxv)
            for p in range(8):
                pltpu.async_copy(parts[p].at[idxv], tmp.at[p], sem)
            for p in range(8):
                pltpu.make_async_copy(parts[p].at[idxv], tmp.at[p], sem).wait()
            for p in range(8):
                @pl.loop(0, per_tile)
                def _cp(i, p=p):
                    buf[i, pl.ds(p * 16, 16)] = tmp[p, i, :]
            pltpu.sync_copy(buf, out.at[jset].at[pl.ds(wid * per_tile, per_tile)])

    return gather_cat


def _tc_layer_packed(agg_pk, w, b):
    npk = agg_pk.shape[1]
    eye8 = jnp.eye(8, dtype=jnp.float32)
    waa = jnp.kron(eye8, w[:16, :16])
    wba = jnp.kron(eye8, w[16:, :16])
    wab = jnp.kron(eye8, w[:16, 16:])
    wbb = jnp.kron(eye8, w[16:, 16:])
    bias_a = jnp.tile(b[:16], 8).reshape(1, 128)
    bias_b = jnp.tile(b[16:], 8).reshape(1, 128)

    r = npk // 4
    assert r % 8 == 0

    def body(a_ref, b_ref, waa_r, wba_r, wab_r, wbb_r, ba_r, bb_r,
             oa_ref, ob_ref):
        a = a_ref[0]
        bm = b_ref[0]
        ha = (jnp.dot(a, waa_r[...], preferred_element_type=jnp.float32)
              + jnp.dot(bm, wba_r[...], preferred_element_type=jnp.float32))
        hb = (jnp.dot(a, wab_r[...], preferred_element_type=jnp.float32)
              + jnp.dot(bm, wbb_r[...], preferred_element_type=jnp.float32))
        oa_ref[...] = jnp.maximum(ha + ba_r[...], 0.0)
        ob_ref[...] = jnp.maximum(hb + bb_r[...], 0.0)

    def wspec(i):
        return pl.BlockSpec((128, 128), lambda i: (0, 0))

    ha, hb = pl.pallas_call(
        body,
        grid=(4,),
        in_specs=[
            pl.BlockSpec((1, r, 128), lambda i: (0, i, 0)),
            pl.BlockSpec((1, r, 128), lambda i: (1, i, 0)),
            wspec(0), wspec(1), wspec(2), wspec(3),
            pl.BlockSpec((1, 128), lambda i: (0, 0)),
            pl.BlockSpec((1, 128), lambda i: (0, 0)),
        ],
        out_specs=[
            pl.BlockSpec((r, 128), lambda i: (i, 0)),
            pl.BlockSpec((r, 128), lambda i: (i, 0)),
        ],
        out_shape=[jax.ShapeDtypeStruct((npk, 128), jnp.float32)] * 2,
        name="tc_gnn_layer",
    )(agg_pk, agg_pk, waa, wba, wab, wbb, bias_a, bias_b)
    return ha, hb


def _tc_mlp(ug, sig, tig, ws, bs, wt, bt, dws, dbs, dwt, dbt):
    bsz = ug.shape[0]
    rb = 512
    grid = bsz // rb

    def body(u_ref, s_ref, t_ref,
             ws0, ws1, ws2, bs0, bs1, bs2,
             wt0, wt1, wt2, bt0, bt1, bt2,
             dws_ref, dbs_ref, dwt_ref, dbt_ref, out_ref):
        u = u_ref[...]
        xs = jnp.concatenate([u, s_ref[...]], axis=1)
        xt = jnp.concatenate([u, t_ref[...]], axis=1)
        for wr, br in ((ws0, bs0), (ws1, bs1), (ws2, bs2)):
            xs = jnp.maximum(
                jnp.dot(xs, wr[...], preferred_element_type=jnp.float32) + br[...], 0.0)
        for wr, br in ((wt0, bt0), (wt1, bt1), (wt2, bt2)):
            xt = jnp.maximum(
                jnp.dot(xt, wr[...], preferred_element_type=jnp.float32) + br[...], 0.0)
        ss = jnp.dot(xs, dws_ref[...], preferred_element_type=jnp.float32) + dbs_ref[...]
        st = jnp.dot(xt, dwt_ref[...], preferred_element_type=jnp.float32) + dbt_ref[...]
        out_ref[...] = jnp.concatenate([ss, st], axis=1)

    def wspec(shape):
        return pl.BlockSpec(shape, lambda i: tuple(0 for _ in shape))

    in_specs = [pl.BlockSpec((rb, 128), lambda i: (i, 0))] * 3
    in_specs += [wspec(w.shape) for w in ws]
    in_specs += [wspec(b.shape) for b in bs]
    in_specs += [wspec(w.shape) for w in wt]
    in_specs += [wspec(b.shape) for b in bt]
    in_specs += [wspec(dws.shape), wspec(dbs.shape), wspec(dwt.shape), wspec(dbt.shape)]

    return pl.pallas_call(
        body,
        grid=(grid,),
        in_specs=in_specs,
        out_specs=pl.BlockSpec((rb, 2), lambda i: (i, 0)),
        out_shape=jax.ShapeDtypeStruct((bsz, 2), jnp.float32),
        name="tc_ncf_mlp",
    )(ug, sig, tig, *ws, *bs, *wt, *bt, dws, dbs, dwt, dbt)


def kernel(params, edge_weight, u, si, ti, edge_index):
    user_n = params["user_emb"].shape[0]
    i1_n = params["item_s_emb"].shape[0]
    n_nodes = user_n + i1_n + params["item_t_emb"].shape[0]
    e = edge_index.shape[1]
    bsz = u.shape[0]

    rows_real = e // _STREAM
    rpt_e = -(-rows_real // _NS)
    rpt_e = -(-rpt_e // (2 * _KCH)) * (2 * _KCH)
    groups = rpt_e // _KCH
    ep2 = jnp.swapaxes(edge_index.reshape(2, -1, _STREAM), 0, 1)
    wp = jnp.concatenate(
        [edge_weight,
         jnp.zeros((rpt_e * _NS * _STREAM - e,), edge_weight.dtype)]
    ).reshape(-1, _STREAM)

    n_pad = -(-n_nodes // (_NS * 8)) * (_NS * 8)
    npk = n_pad // 8

    ego = jnp.concatenate(
        [params["user_emb"], params["item_s_emb"], params["item_t_emb"],
         jnp.zeros((n_pad - n_nodes, 32), jnp.float32)], axis=0)
    ego8 = ego.reshape(npk, 8, 32)
    h_pa = ego8[:, :, :16].reshape(npk, 128)
    h_pb = ego8[:, :, 16:].reshape(npk, 128)

    zeros_n16 = jnp.zeros((n_pad, 16), jnp.float32)
    msgpass = _build_msgpass(n_pad, groups, rows_real)

    parts = [h_pa.reshape(n_pad, 16), h_pb.reshape(n_pad, 16)]
    for k in range(len(params["gnn_W"])):
        agg = msgpass(parts[-2], parts[-1], zeros_n16, ep2, wp)
        h_pa, h_pb = _tc_layer_packed(agg.reshape(2, npk, 128),
                                      params["gnn_W"][k], params["gnn_b"][k])
        parts += [h_pa.reshape(n_pad, 16), h_pb.reshape(n_pad, 16)]

    gidx_flat = jnp.stack([u, si + user_n, ti + user_n + i1_n]).reshape(-1)
    gath = _build_gather_cat(n_pad, bsz)(*parts, gidx_flat)

    return _tc_mlp(
        gath[0], gath[1], gath[2],
        params["ncf_s_W"], [b.reshape(1, -1) for b in params["ncf_s_b"]],
        params["ncf_t_W"], [b.reshape(1, -1) for b in params["ncf_t_b"]],
        params["dense_s_W"], params["dense_s_b"].reshape(1, 1),
        params["dense_t_W"], params["dense_t_b"].reshape(1, 1),
    )
